# Initial kernel scaffold; baseline (speedup 1.0000x reference)
import functools

import jax
import jax.numpy as jnp
from jax import lax
from jax.experimental import pallas as pl
from jax.experimental.pallas import tpu as pltpu
from jax.experimental.pallas import tpu_sc as plsc

N_U = 10000
D = 128
E_REL = 320000
CHUNK = 128
NS = 16
NC = 2
CH_PER_CORE = E_REL // CHUNK
CH_MAIN = CH_PER_CORE // NS
CH_EXTRA = CH_PER_CORE - CH_MAIN * NS
ROWS_PER_TILE = N_U // NS
DEG_W = 16


def _sc_aggregate(x_stack, src_all, dst_all):
    mesh = plsc.VectorSubcoreMesh(core_axis_name="c", subcore_axis_name="s")

    @functools.partial(
        pl.kernel,
        out_type=(
            jax.ShapeDtypeStruct((NC * N_U, D), jnp.float32),
            jax.ShapeDtypeStruct((NC * N_U, DEG_W), jnp.float32),
        ),
        mesh=mesh,
        scratch_types=[
            pltpu.VMEM((CHUNK,), jnp.int32),
            pltpu.VMEM((CHUNK,), jnp.int32),
            pltpu.VMEM((CHUNK, D), jnp.float32),
            pltpu.VMEM((CHUNK, DEG_W), jnp.float32),
            pltpu.VMEM((ROWS_PER_TILE, DEG_W), jnp.float32),
            pltpu.VMEM_SHARED((N_U, D), jnp.float32),
            pltpu.VMEM_SHARED((N_U, DEG_W), jnp.float32),
            pltpu.SemaphoreType.DMA,
        ],
    )
    def k(x_hbm, src_hbm, dst_hbm, agg_hbm, deg_hbm,
          src_v, dst_v, rows_v, ones_v, zdeg_v, acc_sh, deg_sh, sem):
        c = lax.axis_index("c")
        s = lax.axis_index("s")

        def init_ones(i, carry):
            ones_v[i, :] = jnp.full((DEG_W,), 1.0, jnp.float32)
            return carry
        lax.fori_loop(0, CHUNK, init_ones, 0)

        def init_zdeg(i, carry):
            zdeg_v[i, :] = jnp.zeros((DEG_W,), jnp.float32)
            return carry
        lax.fori_loop(0, ROWS_PER_TILE, init_zdeg, 0)

        def init_rows(i, carry):
            rows_v[i // 8, pl.ds((i % 8) * 16, 16)] = jnp.zeros((16,), jnp.float32)
            return carry
        lax.fori_loop(0, CHUNK * 8, init_rows, 0)

        row0 = s * ROWS_PER_TILE
        for kk in range(ROWS_PER_TILE // CHUNK):
            pltpu.sync_copy(rows_v, acc_sh.at[pl.ds(row0 + kk * CHUNK, CHUNK)])
        rem = ROWS_PER_TILE % CHUNK
        pltpu.sync_copy(rows_v.at[pl.ds(0, rem)],
                        acc_sh.at[pl.ds(row0 + ROWS_PER_TILE - rem, rem)])
        pltpu.sync_copy(zdeg_v, deg_sh.at[pl.ds(row0, ROWS_PER_TILE)])
        plsc.subcore_barrier()

        base_ch = c * CH_PER_CORE + s * CH_MAIN

        def do_chunk(ch):
            ebase = ch * CHUNK
            pltpu.sync_copy(src_hbm.at[pl.ds(ebase, CHUNK)], src_v)
            pltpu.sync_copy(dst_hbm.at[pl.ds(ebase, CHUNK)], dst_v)
            pltpu.async_copy(x_hbm.at[src_v], rows_v, sem).wait()
            pltpu.sync_copy(rows_v, acc_sh.at[dst_v], add=True)
            pltpu.sync_copy(ones_v, deg_sh.at[dst_v], add=True)

        def body(g, carry):
            do_chunk(base_ch + g)
            return carry
        lax.fori_loop(0, CH_MAIN, body, 0)

        @pl.when(s < CH_EXTRA)
        def _():
            do_chunk(c * CH_PER_CORE + CH_MAIN * NS + s)

        plsc.subcore_barrier()

        out0 = c * N_U + row0
        pltpu.sync_copy(acc_sh.at[pl.ds(row0, ROWS_PER_TILE)],
                        agg_hbm.at[pl.ds(out0, ROWS_PER_TILE)])
        pltpu.sync_copy(deg_sh.at[pl.ds(row0, ROWS_PER_TILE)],
                        deg_hbm.at[pl.ds(out0, ROWS_PER_TILE)])

    return k(x_stack, src_all, dst_all)


def _tc_finish(aggs, degs, w_f, w_b):
    BR = 1000
    nblk = N_U // BR

    def body(a0_ref, a1_ref, d0_ref, d1_ref, w0_ref, w1_ref, o_ref):
        d0 = jnp.maximum(d0_ref[...][:, 0:1], 1.0)
        d1 = jnp.maximum(d1_ref[...][:, 0:1], 1.0)
        a0 = a0_ref[...] / d0
        a1 = a1_ref[...] / d1
        o_ref[...] = (
            jnp.dot(a0, w0_ref[...], preferred_element_type=jnp.float32)
            + jnp.dot(a1, w1_ref[...], preferred_element_type=jnp.float32)
        )

    return pl.pallas_call(
        body,
        grid=(nblk,),
        in_specs=[
            pl.BlockSpec((BR, D), lambda i: (i, 0)),
            pl.BlockSpec((BR, D), lambda i: (i + nblk, 0)),
            pl.BlockSpec((BR, DEG_W), lambda i: (i, 0)),
            pl.BlockSpec((BR, DEG_W), lambda i: (i + nblk, 0)),
            pl.BlockSpec((D, D), lambda i: (0, 0)),
            pl.BlockSpec((D, D), lambda i: (0, 0)),
        ],
        out_specs=pl.BlockSpec((BR, D), lambda i: (i, 0)),
        out_shape=jax.ShapeDtypeStruct((N_U, D), jnp.float32),
    )(aggs, aggs, degs, degs, w_f, w_b)


def kernel(x_user, x_item, edge_index_follows, edge_index_bought,
           W_follows, W_bought):
    src_f = edge_index_follows[0].astype(jnp.int32)
    dst_f = edge_index_follows[1].astype(jnp.int32)
    src_b = edge_index_bought[0].astype(jnp.int32) + N_U
    dst_b = edge_index_bought[1].astype(jnp.int32)
    x_stack = jnp.concatenate([x_user, x_item], axis=0)
    src_all = jnp.concatenate([src_f, src_b])
    dst_all = jnp.concatenate([dst_f, dst_b])
    aggs, degs = _sc_aggregate(x_stack, src_all, dst_all)
    return _tc_finish(aggs, degs, W_follows, W_bought)


# speedup vs baseline: 6.4332x; 6.4332x over previous
import functools

import jax
import jax.numpy as jnp
from jax import lax
from jax.experimental import pallas as pl
from jax.experimental.pallas import tpu as pltpu
from jax.experimental.pallas import tpu_sc as plsc

N_U = 10000
D = 128
E_REL = 320000
CHUNK = 128
NS = 16
NC = 2
CH_PER_CORE = E_REL // CHUNK
CH_MAIN = CH_PER_CORE // NS
CH_EXTRA = CH_PER_CORE - CH_MAIN * NS
R_MAIN = 624
R_TAIL = N_U - R_MAIN * NS
DEG_W = 16


def _sc_aggregate(x_stack, src_all, dst_all):
    mesh = plsc.VectorSubcoreMesh(core_axis_name="c", subcore_axis_name="s")

    @functools.partial(
        pl.kernel,
        out_type=(
            jax.ShapeDtypeStruct((NC * N_U, D), jnp.float32),
            jax.ShapeDtypeStruct((NC * N_U,), jnp.float32),
        ),
        mesh=mesh,
        scratch_types=[
            pltpu.VMEM((CHUNK,), jnp.int32),
            pltpu.VMEM((CHUNK,), jnp.int32),
            pltpu.VMEM((CHUNK, D), jnp.float32),
            pltpu.VMEM((CHUNK,), jnp.float32),
            pltpu.VMEM((R_MAIN,), jnp.float32),
            pltpu.VMEM_SHARED((N_U, D), jnp.float32),
            pltpu.VMEM_SHARED((N_U,), jnp.float32),
            pltpu.SemaphoreType.DMA,
        ],
    )
    def k(x_hbm, src_hbm, dst_hbm, agg_hbm, deg_hbm,
          src_v, dst_v, rows_v, ones_v, zdeg_v, acc_sh, deg_sh, sem):
        c = lax.axis_index("c")
        s = lax.axis_index("s")

        def init_ones(i, carry):
            ones_v[pl.ds(i * 16, 16)] = jnp.full((16,), 1.0, jnp.float32)
            return carry
        lax.fori_loop(0, CHUNK // 16, init_ones, 0)

        def init_zdeg(i, carry):
            zdeg_v[pl.ds(i * 16, 16)] = jnp.zeros((16,), jnp.float32)
            return carry
        lax.fori_loop(0, R_MAIN // 16, init_zdeg, 0)

        def init_rows(i, carry):
            rows_v[i // 8, pl.ds((i % 8) * 16, 16)] = jnp.zeros((16,), jnp.float32)
            return carry
        lax.fori_loop(0, CHUNK * 8, init_rows, 0)

        row0 = s * R_MAIN
        for kk in range(R_MAIN // CHUNK):
            pltpu.sync_copy(rows_v, acc_sh.at[pl.ds(row0 + kk * CHUNK, CHUNK)])
        rem = R_MAIN % CHUNK
        pltpu.sync_copy(rows_v.at[pl.ds(0, rem)],
                        acc_sh.at[pl.ds(row0 + R_MAIN - rem, rem)])
        pltpu.sync_copy(zdeg_v, deg_sh.at[pl.ds(row0, R_MAIN)])

        @pl.when(s == NS - 1)
        def _():
            pltpu.sync_copy(rows_v.at[pl.ds(0, R_TAIL)],
                            acc_sh.at[pl.ds(N_U - R_TAIL, R_TAIL)])
            pltpu.sync_copy(zdeg_v.at[pl.ds(0, R_TAIL)],
                            deg_sh.at[pl.ds(N_U - R_TAIL, R_TAIL)])

        plsc.subcore_barrier()

        base_ch = c * CH_PER_CORE + s * CH_MAIN

        def do_chunk(ch):
            ebase = ch * CHUNK
            pltpu.sync_copy(src_hbm.at[pl.ds(ebase, CHUNK)], src_v)
            pltpu.sync_copy(dst_hbm.at[pl.ds(ebase, CHUNK)], dst_v)
            pltpu.async_copy(x_hbm.at[src_v], rows_v, sem).wait()
            pltpu.sync_copy(rows_v, acc_sh.at[dst_v], add=True)
            pltpu.sync_copy(ones_v, deg_sh.at[dst_v], add=True)

        def body(g, carry):
            do_chunk(base_ch + g)
            return carry
        lax.fori_loop(0, CH_MAIN, body, 0)

        @pl.when(s < CH_EXTRA)
        def _():
            do_chunk(c * CH_PER_CORE + CH_MAIN * NS + s)

        plsc.subcore_barrier()

        out0 = c * N_U + row0
        pltpu.sync_copy(acc_sh.at[pl.ds(row0, R_MAIN)],
                        agg_hbm.at[pl.ds(out0, R_MAIN)])
        pltpu.sync_copy(deg_sh.at[pl.ds(row0, R_MAIN)], zdeg_v)
        pltpu.sync_copy(zdeg_v, deg_hbm.at[pl.ds(out0, R_MAIN)])

        @pl.when(s == NS - 1)
        def _():
            pltpu.sync_copy(acc_sh.at[pl.ds(N_U - R_TAIL, R_TAIL)],
                            agg_hbm.at[pl.ds(c * N_U + N_U - R_TAIL, R_TAIL)])
            pltpu.sync_copy(deg_sh.at[pl.ds(N_U - R_TAIL, R_TAIL)],
                            ones_v.at[pl.ds(0, R_TAIL)])
            pltpu.sync_copy(ones_v.at[pl.ds(0, R_TAIL)],
                            deg_hbm.at[pl.ds(c * N_U + N_U - R_TAIL, R_TAIL)])

    return k(x_stack, src_all, dst_all)


def _tc_finish(aggs, degs, w_f, w_b):
    BR = 1000
    nblk = N_U // BR
    degs2 = degs.reshape(NC * N_U, 1)

    def body(a0_ref, a1_ref, d0_ref, d1_ref, w0_ref, w1_ref, o_ref):
        d0 = jnp.maximum(d0_ref[...], 1.0)
        d1 = jnp.maximum(d1_ref[...], 1.0)
        a0 = a0_ref[...] / d0
        a1 = a1_ref[...] / d1
        o_ref[...] = (
            jnp.dot(a0, w0_ref[...], preferred_element_type=jnp.float32)
            + jnp.dot(a1, w1_ref[...], preferred_element_type=jnp.float32)
        )

    return pl.pallas_call(
        body,
        grid=(nblk,),
        in_specs=[
            pl.BlockSpec((BR, D), lambda i: (i, 0)),
            pl.BlockSpec((BR, D), lambda i: (i + nblk, 0)),
            pl.BlockSpec((BR, 1), lambda i: (i, 0)),
            pl.BlockSpec((BR, 1), lambda i: (i + nblk, 0)),
            pl.BlockSpec((D, D), lambda i: (0, 0)),
            pl.BlockSpec((D, D), lambda i: (0, 0)),
        ],
        out_specs=pl.BlockSpec((BR, D), lambda i: (i, 0)),
        out_shape=jax.ShapeDtypeStruct((N_U, D), jnp.float32),
    )(aggs, aggs, degs2, degs2, w_f, w_b)


def kernel(x_user, x_item, edge_index_follows, edge_index_bought,
           W_follows, W_bought):
    src_f = edge_index_follows[0].astype(jnp.int32)
    dst_f = edge_index_follows[1].astype(jnp.int32)
    src_b = edge_index_bought[0].astype(jnp.int32) + N_U
    dst_b = edge_index_bought[1].astype(jnp.int32)
    x_stack = jnp.concatenate([x_user, x_item], axis=0)
    src_all = jnp.concatenate([src_f, src_b])
    dst_all = jnp.concatenate([dst_f, dst_b])
    aggs, degs = _sc_aggregate(x_stack, src_all, dst_all)
    return _tc_finish(aggs, degs, W_follows, W_bought)


# speedup vs baseline: 11.3390x; 1.7626x over previous
import functools

import jax
import jax.numpy as jnp
from jax import lax
from jax.experimental import pallas as pl
from jax.experimental.pallas import tpu as pltpu
from jax.experimental.pallas import tpu_sc as plsc

N_U = 10000
D = 128
E_REL = 320000
CHUNK = 128
NS = 16
NC = 2
CH_PER_CORE = E_REL // CHUNK
CH_MAIN = CH_PER_CORE // NS
CH_EXTRA = CH_PER_CORE - CH_MAIN * NS
NROW = 2
NIDX = 4
R_MAIN = 624
R_TAIL = N_U - R_MAIN * NS
DEG_W = 16


def _sc_aggregate(x_stack, src_all, dst_all):
    mesh = plsc.VectorSubcoreMesh(core_axis_name="c", subcore_axis_name="s")

    @functools.partial(
        pl.kernel,
        out_type=(
            jax.ShapeDtypeStruct((NC * N_U, D), jnp.float32),
            jax.ShapeDtypeStruct((NC * N_U,), jnp.float32),
        ),
        mesh=mesh,
        scratch_types=[
            [pltpu.VMEM((CHUNK,), jnp.int32) for _ in range(NIDX)],
            [pltpu.VMEM((CHUNK,), jnp.int32) for _ in range(NIDX)],
            [pltpu.VMEM((CHUNK, D), jnp.float32) for _ in range(NROW)],
            pltpu.VMEM((CHUNK,), jnp.float32),
            pltpu.VMEM((R_MAIN,), jnp.float32),
            pltpu.VMEM_SHARED((N_U, D), jnp.float32),
            pltpu.VMEM_SHARED((N_U,), jnp.float32),
            [pltpu.SemaphoreType.DMA for _ in range(NIDX)],
            [pltpu.SemaphoreType.DMA for _ in range(NROW)],
            [pltpu.SemaphoreType.DMA for _ in range(NROW)],
        ],
    )
    def k(x_hbm, src_hbm, dst_hbm, agg_hbm, deg_hbm,
          src_v, dst_v, rows_v, ones_v, zdeg_v, acc_sh, deg_sh,
          si, sg, ss):
        c = lax.axis_index("c")
        s = lax.axis_index("s")

        def init_ones(i, carry):
            ones_v[pl.ds(i * 16, 16)] = jnp.full((16,), 1.0, jnp.float32)
            return carry
        lax.fori_loop(0, CHUNK // 16, init_ones, 0)

        def init_zdeg(i, carry):
            zdeg_v[pl.ds(i * 16, 16)] = jnp.zeros((16,), jnp.float32)
            return carry
        lax.fori_loop(0, R_MAIN // 16, init_zdeg, 0)

        def init_rows(i, carry):
            rows_v[0][i // 8, pl.ds((i % 8) * 16, 16)] = jnp.zeros((16,), jnp.float32)
            return carry
        lax.fori_loop(0, CHUNK * 8, init_rows, 0)

        row0 = s * R_MAIN
        for kk in range(R_MAIN // CHUNK):
            pltpu.sync_copy(rows_v[0], acc_sh.at[pl.ds(row0 + kk * CHUNK, CHUNK)])
        rem = R_MAIN % CHUNK
        pltpu.sync_copy(rows_v[0].at[pl.ds(0, rem)],
                        acc_sh.at[pl.ds(row0 + R_MAIN - rem, rem)])
        pltpu.sync_copy(zdeg_v, deg_sh.at[pl.ds(row0, R_MAIN)])

        @pl.when(s == NS - 1)
        def _():
            pltpu.sync_copy(rows_v[0].at[pl.ds(0, R_TAIL)],
                            acc_sh.at[pl.ds(N_U - R_TAIL, R_TAIL)])
            pltpu.sync_copy(zdeg_v.at[pl.ds(0, R_TAIL)],
                            deg_sh.at[pl.ds(N_U - R_TAIL, R_TAIL)])

        plsc.subcore_barrier()

        base_ch = c * CH_PER_CORE + s * CH_MAIN

        def ebase(g):
            return (base_ch + g) * CHUNK

        def issue_idx(g, r):
            pltpu.async_copy(src_hbm.at[pl.ds(ebase(g), CHUNK)], src_v[r], si[r])
            pltpu.async_copy(dst_hbm.at[pl.ds(ebase(g), CHUNK)], dst_v[r], si[r])

        def wait_idx(g, r):
            pltpu.make_async_copy(src_hbm.at[pl.ds(ebase(g), CHUNK)],
                                  src_v[r], si[r]).wait()
            pltpu.make_async_copy(dst_hbm.at[pl.ds(ebase(g), CHUNK)],
                                  dst_v[r], si[r]).wait()

        def issue_gather(r, p):
            pltpu.async_copy(x_hbm.at[src_v[r]], rows_v[p], sg[p])

        def wait_gather(r, p):
            pltpu.make_async_copy(x_hbm.at[src_v[r]], rows_v[p], sg[p]).wait()

        def issue_scatter(r, p):
            pltpu.async_copy(rows_v[p], acc_sh.at[dst_v[r]], ss[p], add=True)
            pltpu.async_copy(ones_v, deg_sh.at[dst_v[r]], ss[p], add=True)

        def wait_scatter(r, p):
            pltpu.make_async_copy(rows_v[p], acc_sh.at[dst_v[r]], ss[p]).wait()
            pltpu.make_async_copy(ones_v, deg_sh.at[dst_v[r]], ss[p]).wait()

        def step(g, gm2, gm4, drain_sc=True, pf_idx=True, issue_g=True):
            p, r = gm2, gm4
            q = 1 - p
            r1 = (gm4 + 1) % NIDX
            r3 = (gm4 + 3) % NIDX
            wait_gather(r, p)
            issue_scatter(r, p)
            if drain_sc:
                wait_scatter(r3, q)
            if issue_g:
                wait_idx(g + 1, r1)
                issue_gather(r1, q)
            if pf_idx:
                issue_idx(g + 3, r3)

        issue_idx(0, 0)
        issue_idx(1, 1)
        issue_idx(2, 2)
        wait_idx(0, 0)
        issue_gather(0, 0)
        step(0, 0, 0, drain_sc=False)
        step(1, 1, 1)

        def body(i, carry):
            g0 = 2 + i * 4
            for j in range(4):
                step(g0 + j, (2 + j) % NROW, (2 + j) % NIDX)
            return carry
        lax.fori_loop(0, (CH_MAIN - 8) // 4, body, 0)

        for g in range(CH_MAIN - 6, CH_MAIN):
            step(g, g % NROW, g % NIDX,
                 pf_idx=(g + 3 < CH_MAIN), issue_g=(g + 1 < CH_MAIN))
        wait_scatter((CH_MAIN - 1) % NIDX, (CH_MAIN - 1) % NROW)

        @pl.when(s < CH_EXTRA)
        def _():
            ch = c * CH_PER_CORE + CH_MAIN * NS + s
            pltpu.sync_copy(src_hbm.at[pl.ds(ch * CHUNK, CHUNK)], src_v[0])
            pltpu.sync_copy(dst_hbm.at[pl.ds(ch * CHUNK, CHUNK)], dst_v[0])
            pltpu.async_copy(x_hbm.at[src_v[0]], rows_v[0], sg[0]).wait()
            pltpu.sync_copy(rows_v[0], acc_sh.at[dst_v[0]], add=True)
            pltpu.sync_copy(ones_v, deg_sh.at[dst_v[0]], add=True)

        plsc.subcore_barrier()

        out0 = c * N_U + row0
        pltpu.sync_copy(acc_sh.at[pl.ds(row0, R_MAIN)],
                        agg_hbm.at[pl.ds(out0, R_MAIN)])
        pltpu.sync_copy(deg_sh.at[pl.ds(row0, R_MAIN)], zdeg_v)
        pltpu.sync_copy(zdeg_v, deg_hbm.at[pl.ds(out0, R_MAIN)])

        @pl.when(s == NS - 1)
        def _():
            pltpu.sync_copy(acc_sh.at[pl.ds(N_U - R_TAIL, R_TAIL)],
                            agg_hbm.at[pl.ds(c * N_U + N_U - R_TAIL, R_TAIL)])
            pltpu.sync_copy(deg_sh.at[pl.ds(N_U - R_TAIL, R_TAIL)],
                            ones_v.at[pl.ds(0, R_TAIL)])
            pltpu.sync_copy(ones_v.at[pl.ds(0, R_TAIL)],
                            deg_hbm.at[pl.ds(c * N_U + N_U - R_TAIL, R_TAIL)])

    return k(x_stack, src_all, dst_all)


def _tc_finish(aggs, degs, w_f, w_b):
    BR = 1000
    nblk = N_U // BR
    degs2 = degs.reshape(NC * N_U, 1)

    def body(a0_ref, a1_ref, d0_ref, d1_ref, w0_ref, w1_ref, o_ref):
        d0 = jnp.maximum(d0_ref[...], 1.0)
        d1 = jnp.maximum(d1_ref[...], 1.0)
        a0 = a0_ref[...] / d0
        a1 = a1_ref[...] / d1
        o_ref[...] = (
            jnp.dot(a0, w0_ref[...], preferred_element_type=jnp.float32)
            + jnp.dot(a1, w1_ref[...], preferred_element_type=jnp.float32)
        )

    return pl.pallas_call(
        body,
        grid=(nblk,),
        in_specs=[
            pl.BlockSpec((BR, D), lambda i: (i, 0)),
            pl.BlockSpec((BR, D), lambda i: (i + nblk, 0)),
            pl.BlockSpec((BR, 1), lambda i: (i, 0)),
            pl.BlockSpec((BR, 1), lambda i: (i + nblk, 0)),
            pl.BlockSpec((D, D), lambda i: (0, 0)),
            pl.BlockSpec((D, D), lambda i: (0, 0)),
        ],
        out_specs=pl.BlockSpec((BR, D), lambda i: (i, 0)),
        out_shape=jax.ShapeDtypeStruct((N_U, D), jnp.float32),
    )(aggs, aggs, degs2, degs2, w_f, w_b)


def kernel(x_user, x_item, edge_index_follows, edge_index_bought,
           W_follows, W_bought):
    src_f = edge_index_follows[0].astype(jnp.int32)
    dst_f = edge_index_follows[1].astype(jnp.int32)
    src_b = edge_index_bought[0].astype(jnp.int32) + N_U
    dst_b = edge_index_bought[1].astype(jnp.int32)
    x_stack = jnp.concatenate([x_user, x_item], axis=0)
    src_all = jnp.concatenate([src_f, src_b])
    dst_all = jnp.concatenate([dst_f, dst_b])
    aggs, degs = _sc_aggregate(x_stack, src_all, dst_all)
    return _tc_finish(aggs, degs, W_follows, W_bought)


# speedup vs baseline: 12.4335x; 1.0965x over previous
import functools

import jax
import jax.numpy as jnp
from jax import lax
from jax.experimental import pallas as pl
from jax.experimental.pallas import tpu as pltpu
from jax.experimental.pallas import tpu_sc as plsc

N_U = 10000
D = 128
E_REL = 320000
CHUNK = 80
NS = 16
NC = 2
CH_PER_CORE = E_REL // CHUNK
CH_MAIN = CH_PER_CORE // NS
NROW = 4
NIDX = 8
R_MAIN = 624
R_TAIL = N_U - R_MAIN * NS


def _sc_aggregate(x_stack, src_all, dst_all):
    mesh = plsc.VectorSubcoreMesh(core_axis_name="c", subcore_axis_name="s")

    @functools.partial(
        pl.kernel,
        out_type=(
            jax.ShapeDtypeStruct((NC * N_U, D), jnp.float32),
            jax.ShapeDtypeStruct((NC * N_U,), jnp.float32),
        ),
        mesh=mesh,
        scratch_types=[
            [pltpu.VMEM((CHUNK,), jnp.int32) for _ in range(NIDX)],
            [pltpu.VMEM((CHUNK,), jnp.int32) for _ in range(NIDX)],
            [pltpu.VMEM((CHUNK, D), jnp.float32) for _ in range(NROW)],
            pltpu.VMEM((CHUNK,), jnp.float32),
            pltpu.VMEM((R_MAIN,), jnp.float32),
            pltpu.VMEM_SHARED((N_U, D), jnp.float32),
            pltpu.VMEM_SHARED((N_U,), jnp.float32),
            [pltpu.SemaphoreType.DMA for _ in range(NIDX)],
            [pltpu.SemaphoreType.DMA for _ in range(NROW)],
            [pltpu.SemaphoreType.DMA for _ in range(NROW)],
        ],
    )
    def k(x_hbm, src_hbm, dst_hbm, agg_hbm, deg_hbm,
          src_v, dst_v, rows_v, ones_v, zdeg_v, acc_sh, deg_sh,
          si, sg, ss):
        c = lax.axis_index("c")
        s = lax.axis_index("s")

        def init_ones(i, carry):
            ones_v[pl.ds(i * 16, 16)] = jnp.full((16,), 1.0, jnp.float32)
            return carry
        lax.fori_loop(0, CHUNK // 16, init_ones, 0)

        def init_zdeg(i, carry):
            zdeg_v[pl.ds(i * 16, 16)] = jnp.zeros((16,), jnp.float32)
            return carry
        lax.fori_loop(0, R_MAIN // 16, init_zdeg, 0)

        def init_rows(i, carry):
            rows_v[0][i // 8, pl.ds((i % 8) * 16, 16)] = jnp.zeros((16,), jnp.float32)
            return carry
        lax.fori_loop(0, CHUNK * 8, init_rows, 0)

        row0 = s * R_MAIN
        for kk in range(R_MAIN // CHUNK):
            pltpu.sync_copy(rows_v[0], acc_sh.at[pl.ds(row0 + kk * CHUNK, CHUNK)])
        rem = R_MAIN % CHUNK
        pltpu.sync_copy(rows_v[0].at[pl.ds(0, rem)],
                        acc_sh.at[pl.ds(row0 + R_MAIN - rem, rem)])
        pltpu.sync_copy(zdeg_v, deg_sh.at[pl.ds(row0, R_MAIN)])

        @pl.when(s == NS - 1)
        def _():
            pltpu.sync_copy(rows_v[0].at[pl.ds(0, R_TAIL)],
                            acc_sh.at[pl.ds(N_U - R_TAIL, R_TAIL)])
            pltpu.sync_copy(zdeg_v.at[pl.ds(0, R_TAIL)],
                            deg_sh.at[pl.ds(N_U - R_TAIL, R_TAIL)])

        plsc.subcore_barrier()

        base_ch = c * CH_PER_CORE + s * CH_MAIN

        def ebase(g):
            return (base_ch + g) * CHUNK

        def issue_idx(g, r):
            pltpu.async_copy(src_hbm.at[pl.ds(ebase(g), CHUNK)], src_v[r], si[r])
            pltpu.async_copy(dst_hbm.at[pl.ds(ebase(g), CHUNK)], dst_v[r], si[r])

        def wait_idx(g, r):
            pltpu.make_async_copy(src_hbm.at[pl.ds(ebase(g), CHUNK)],
                                  src_v[r], si[r]).wait()
            pltpu.make_async_copy(dst_hbm.at[pl.ds(ebase(g), CHUNK)],
                                  dst_v[r], si[r]).wait()

        def issue_gather(r, p):
            pltpu.async_copy(x_hbm.at[src_v[r]], rows_v[p], sg[p])

        def wait_gather(r, p):
            pltpu.make_async_copy(x_hbm.at[src_v[r]], rows_v[p], sg[p]).wait()

        def issue_scatter(r, p):
            pltpu.async_copy(rows_v[p], acc_sh.at[dst_v[r]], ss[p], add=True)
            pltpu.async_copy(ones_v, deg_sh.at[dst_v[r]], ss[p], add=True)

        def wait_scatter(r, p):
            pltpu.make_async_copy(rows_v[p], acc_sh.at[dst_v[r]], ss[p]).wait()
            pltpu.make_async_copy(ones_v, deg_sh.at[dst_v[r]], ss[p]).wait()

        def step(g, gm4, gm8, drain_sc=True, pf_idx=True, issue_g=True):
            p, r = gm4, gm8
            p2 = (gm4 + 2) % NROW
            r2f = (gm8 + 6) % NIDX
            r2 = (gm8 + 2) % NIDX
            r5 = (gm8 + 5) % NIDX
            wait_gather(r, p)
            issue_scatter(r, p)
            if drain_sc:
                wait_scatter(r2f, p2)
            if issue_g:
                wait_idx(g + 2, r2)
                issue_gather(r2, p2)
            if pf_idx:
                issue_idx(g + 5, r5)

        for g in range(5):
            issue_idx(g, g)
        wait_idx(0, 0)
        issue_gather(0, 0)
        wait_idx(1, 1)
        issue_gather(1, 1)
        step(0, 0, 0, drain_sc=False)
        step(1, 1, 1, drain_sc=False)

        def body(i, carry):
            g0 = 2 + i * 8
            for j in range(8):
                step(g0 + j, (2 + j) % NROW, (2 + j) % NIDX)
            return carry
        lax.fori_loop(0, (CH_MAIN - 10) // 8, body, 0)

        for g in range(CH_MAIN - 8, CH_MAIN):
            step(g, g % NROW, g % NIDX,
                 pf_idx=(g + 5 < CH_MAIN), issue_g=(g + 2 < CH_MAIN))
        wait_scatter((CH_MAIN - 2) % NIDX, (CH_MAIN - 2) % NROW)
        wait_scatter((CH_MAIN - 1) % NIDX, (CH_MAIN - 1) % NROW)

        plsc.subcore_barrier()

        out0 = c * N_U + row0
        pltpu.sync_copy(acc_sh.at[pl.ds(row0, R_MAIN)],
                        agg_hbm.at[pl.ds(out0, R_MAIN)])
        pltpu.sync_copy(deg_sh.at[pl.ds(row0, R_MAIN)], zdeg_v)
        pltpu.sync_copy(zdeg_v, deg_hbm.at[pl.ds(out0, R_MAIN)])

        @pl.when(s == NS - 1)
        def _():
            pltpu.sync_copy(acc_sh.at[pl.ds(N_U - R_TAIL, R_TAIL)],
                            agg_hbm.at[pl.ds(c * N_U + N_U - R_TAIL, R_TAIL)])
            pltpu.sync_copy(deg_sh.at[pl.ds(N_U - R_TAIL, R_TAIL)],
                            ones_v.at[pl.ds(0, R_TAIL)])
            pltpu.sync_copy(ones_v.at[pl.ds(0, R_TAIL)],
                            deg_hbm.at[pl.ds(c * N_U + N_U - R_TAIL, R_TAIL)])

    return k(x_stack, src_all, dst_all)


def _tc_finish(aggs, degs, w_f, w_b):
    BR = 1000
    nblk = N_U // BR
    degs2 = degs.reshape(NC * N_U, 1)

    def body(a0_ref, a1_ref, d0_ref, d1_ref, w0_ref, w1_ref, o_ref):
        d0 = jnp.maximum(d0_ref[...], 1.0)
        d1 = jnp.maximum(d1_ref[...], 1.0)
        a0 = a0_ref[...] / d0
        a1 = a1_ref[...] / d1
        o_ref[...] = (
            jnp.dot(a0, w0_ref[...], preferred_element_type=jnp.float32)
            + jnp.dot(a1, w1_ref[...], preferred_element_type=jnp.float32)
        )

    return pl.pallas_call(
        body,
        grid=(nblk,),
        in_specs=[
            pl.BlockSpec((BR, D), lambda i: (i, 0)),
            pl.BlockSpec((BR, D), lambda i: (i + nblk, 0)),
            pl.BlockSpec((BR, 1), lambda i: (i, 0)),
            pl.BlockSpec((BR, 1), lambda i: (i + nblk, 0)),
            pl.BlockSpec((D, D), lambda i: (0, 0)),
            pl.BlockSpec((D, D), lambda i: (0, 0)),
        ],
        out_specs=pl.BlockSpec((BR, D), lambda i: (i, 0)),
        out_shape=jax.ShapeDtypeStruct((N_U, D), jnp.float32),
    )(aggs, aggs, degs2, degs2, w_f, w_b)


def kernel(x_user, x_item, edge_index_follows, edge_index_bought,
           W_follows, W_bought):
    src_f = edge_index_follows[0].astype(jnp.int32)
    dst_f = edge_index_follows[1].astype(jnp.int32)
    src_b = edge_index_bought[0].astype(jnp.int32) + N_U
    dst_b = edge_index_bought[1].astype(jnp.int32)
    x_stack = jnp.concatenate([x_user, x_item], axis=0)
    src_all = jnp.concatenate([src_f, src_b])
    dst_all = jnp.concatenate([dst_f, dst_b])
    aggs, degs = _sc_aggregate(x_stack, src_all, dst_all)
    return _tc_finish(aggs, degs, W_follows, W_bought)
